# Initial kernel scaffold; baseline (speedup 1.0000x reference)
import functools

import jax
import jax.numpy as jnp
from jax import lax
from jax.experimental import pallas as pl
from jax.experimental.pallas import tpu as pltpu
from jax.experimental.pallas import tpu_sc as plsc

L = 16
NC = 2
NS = 16
NW = NC * NS
CH = 64
G = 128


@functools.lru_cache(maxsize=None)
def _build(N, K):
    assert K == L, "kernel specialised to K == 16"
    NT = -(-N // NW)
    NT = -(-NT // CH) * CH
    NCHUNK = NT // CH
    IDX_PER_CHUNK = CH * K * 2
    NGATHER = IDX_PER_CHUNK // G
    CROWS = IDX_PER_CHUNK // 128
    ROW_W = 2 * K

    mesh = plsc.VectorSubcoreMesh(core_axis_name="c", subcore_axis_name="s")

    @functools.partial(
        pl.kernel,
        mesh=mesh,
        out_type=[jax.ShapeDtypeStruct((N,), jnp.float32)] * 3,
        scratch_types=[
            pltpu.VMEM((CROWS, 128), jnp.int32),
            pltpu.VMEM((IDX_PER_CHUNK, 4), jnp.float32),
            pltpu.VMEM((NT,), jnp.float32),
            pltpu.VMEM((NT,), jnp.float32),
            pltpu.VMEM((NT,), jnp.float32),
            pltpu.SemaphoreType.DMA,
        ],
    )
    def k(table, conn, ox, oy, oz, conn_v, rows_v, bx, by, bz, sem):
        wid = lax.axis_index("s") * NC + lax.axis_index("c")
        base = lax.min(wid * NT, N - NT)
        crow0 = (base * 2 * K) // 128

        iota = lax.iota(jnp.int32, L)
        row_base = iota * ROW_W
        fcol = [jnp.full((L,), f, jnp.int32) for f in range(4)]
        onef = jnp.full((L,), 1.0, jnp.float32)
        zerof = jnp.zeros((L,), jnp.float32)

        def chunk_body(g, carry):
            pltpu.sync_copy(conn.at[pl.ds(crow0 + g * CROWS, CROWS)], conn_v)
            cps = [
                pltpu.async_copy(
                    table.at[conn_v.at[j]],
                    rows_v.at[pl.ds(j * G, G)],
                    sem,
                )
                for j in range(NGATHER)
            ]
            for c in cps:
                c.wait()

            for v in range(CH // L):
                a1 = zerof; b1 = zerof; c1 = zerof
                b2 = zerof; c2 = zerof; c3 = zerof
                d1 = zerof; d2 = zerof; d3 = zerof
                for kk in range(K):
                    r0 = row_base + (v * L * ROW_W + 2 * kk)
                    r1 = r0 + 1
                    x1 = plsc.load_gather(rows_v, [r0, fcol[0]])
                    y1 = plsc.load_gather(rows_v, [r0, fcol[1]])
                    z1 = plsc.load_gather(rows_v, [r0, fcol[2]])
                    u1 = plsc.load_gather(rows_v, [r0, fcol[3]])
                    x2 = plsc.load_gather(rows_v, [r1, fcol[0]])
                    y2 = plsc.load_gather(rows_v, [r1, fcol[1]])
                    z2 = plsc.load_gather(rows_v, [r1, fcol[2]])
                    u2 = plsc.load_gather(rows_v, [r1, fcol[3]])
                    dx = x1 - x2
                    dy = y1 - y2
                    dz = z1 - z2
                    du = u1 - u2
                    r2 = dx * dx + dy * dy + dz * dz
                    w2 = jnp.where(r2 == zerof, onef, onef / r2)
                    wdx = w2 * dx
                    wdy = w2 * dy
                    wdz = w2 * dz
                    wdu = w2 * du
                    a1 = a1 + wdx * dx
                    b1 = b1 + wdx * dy
                    c1 = c1 + wdx * dz
                    b2 = b2 + wdy * dy
                    c2 = c2 + wdy * dz
                    c3 = c3 + wdz * dz
                    d1 = d1 + wdu * dx
                    d2 = d2 + wdu * dy
                    d3 = d3 + wdu * dz

                m11 = b2 * c3 - c2 * c2
                m12 = b1 * c3 - c2 * c1
                m13 = b1 * c2 - b2 * c1
                detA = a1 * m11 - b1 * m12 + c1 * m13
                det1 = d1 * m11 - b1 * (d2 * c3 - c2 * d3) + c1 * (d2 * c2 - b2 * d3)
                det2 = a1 * (d2 * c3 - c2 * d3) - d1 * m12 + c1 * (b1 * d3 - d2 * c1)
                det3 = a1 * (b2 * d3 - c2 * d2) - b1 * (b1 * d3 - c1 * d2) + d1 * m13
                rdet = onef / detA
                off = g * CH + v * L
                bx[pl.ds(off, L)] = det1 * rdet
                by[pl.ds(off, L)] = det2 * rdet
                bz[pl.ds(off, L)] = det3 * rdet
            return carry

        lax.fori_loop(0, NCHUNK, chunk_body, 0)

        pltpu.sync_copy(bx, ox.at[pl.ds(base, NT)])
        pltpu.sync_copy(by, oy.at[pl.ds(base, NT)])
        pltpu.sync_copy(bz, oz.at[pl.ds(base, NT)])

    return k


def kernel(coordinates, u, connectivity_tensor):
    N, K = connectivity_tensor.shape[0], connectivity_tensor.shape[1]
    table = jnp.concatenate([coordinates, u], axis=1)
    conn = connectivity_tensor.astype(jnp.int32).reshape(N * K * 2 // 128, 128)
    ox, oy, oz = _build(N, K)(table, conn)
    return (ox[:, None], oy[:, None], oz[:, None])


# speedup vs baseline: 12.0016x; 12.0016x over previous
import functools

import jax
import jax.numpy as jnp
from jax import lax
from jax.experimental import pallas as pl
from jax.experimental.pallas import tpu as pltpu
from jax.experimental.pallas import tpu_sc as plsc

L = 16
NC = 2
NS = 16
NW = NC * NS
CH = 64
G = 128


@functools.lru_cache(maxsize=None)
def _build(N, K):
    assert K == L, "kernel specialised to K == 16"
    NT = -(-N // NW)
    NT = -(-NT // CH) * CH
    NCHUNK = NT // CH
    IDX_PER_CHUNK = CH * K * 2
    NGATHER = IDX_PER_CHUNK // G
    CROWS = IDX_PER_CHUNK // 128
    ROW_W = 2 * K

    mesh = plsc.VectorSubcoreMesh(core_axis_name="c", subcore_axis_name="s")

    @functools.partial(
        pl.kernel,
        mesh=mesh,
        compiler_params=pltpu.CompilerParams(
            needs_layout_passes=False, use_tc_tiling_on_sc=False
        ),
        out_type=[jax.ShapeDtypeStruct((N,), jnp.float32)] * 3,
        scratch_types=[
            pltpu.VMEM((CROWS, 128), jnp.int32),
            pltpu.VMEM((IDX_PER_CHUNK, 4), jnp.float32),
            pltpu.VMEM((NT,), jnp.float32),
            pltpu.VMEM((NT,), jnp.float32),
            pltpu.VMEM((NT,), jnp.float32),
            pltpu.SemaphoreType.DMA,
        ],
    )
    def k(table, conn, ox, oy, oz, conn_v, rows_v, bx, by, bz, sem):
        wid = lax.axis_index("s") * NC + lax.axis_index("c")
        base = lax.min(wid * NT, N - NT)
        base = pl.multiple_of(base, 32)
        crow0 = (base * 2 * K) // 128
        crow0 = pl.multiple_of(crow0, 8)

        iota = lax.iota(jnp.int32, L)
        row_base = iota * ROW_W
        fcol = [jnp.full((L,), f, jnp.int32) for f in range(4)]
        onef = jnp.full((L,), 1.0, jnp.float32)
        zerof = jnp.zeros((L,), jnp.float32)

        def chunk_body(g, carry):
            crow = pl.multiple_of(crow0 + g * CROWS, 8)
            pltpu.sync_copy(conn.at[pl.ds(crow, CROWS)], conn_v)
            cps = [
                pltpu.async_copy(
                    table.at[conn_v.at[j]],
                    rows_v.at[pl.ds(j * G, G)],
                    sem,
                )
                for j in range(NGATHER)
            ]
            for c in cps:
                c.wait()

            for v in range(CH // L):
                a1 = zerof; b1 = zerof; c1 = zerof
                b2 = zerof; c2 = zerof; c3 = zerof
                d1 = zerof; d2 = zerof; d3 = zerof
                for kk in range(K):
                    r0 = row_base + (v * L * ROW_W + 2 * kk)
                    r1 = r0 + 1
                    x1 = plsc.load_gather(rows_v, [r0, fcol[0]])
                    y1 = plsc.load_gather(rows_v, [r0, fcol[1]])
                    z1 = plsc.load_gather(rows_v, [r0, fcol[2]])
                    u1 = plsc.load_gather(rows_v, [r0, fcol[3]])
                    x2 = plsc.load_gather(rows_v, [r1, fcol[0]])
                    y2 = plsc.load_gather(rows_v, [r1, fcol[1]])
                    z2 = plsc.load_gather(rows_v, [r1, fcol[2]])
                    u2 = plsc.load_gather(rows_v, [r1, fcol[3]])
                    dx = x1 - x2
                    dy = y1 - y2
                    dz = z1 - z2
                    du = u1 - u2
                    r2 = dx * dx + dy * dy + dz * dz
                    w2 = jnp.where(r2 == zerof, onef, onef / r2)
                    wdx = w2 * dx
                    wdy = w2 * dy
                    wdz = w2 * dz
                    wdu = w2 * du
                    a1 = a1 + wdx * dx
                    b1 = b1 + wdx * dy
                    c1 = c1 + wdx * dz
                    b2 = b2 + wdy * dy
                    c2 = c2 + wdy * dz
                    c3 = c3 + wdz * dz
                    d1 = d1 + wdu * dx
                    d2 = d2 + wdu * dy
                    d3 = d3 + wdu * dz

                m11 = b2 * c3 - c2 * c2
                m12 = b1 * c3 - c2 * c1
                m13 = b1 * c2 - b2 * c1
                detA = a1 * m11 - b1 * m12 + c1 * m13
                det1 = d1 * m11 - b1 * (d2 * c3 - c2 * d3) + c1 * (d2 * c2 - b2 * d3)
                det2 = a1 * (d2 * c3 - c2 * d3) - d1 * m12 + c1 * (b1 * d3 - d2 * c1)
                det3 = a1 * (b2 * d3 - c2 * d2) - b1 * (b1 * d3 - c1 * d2) + d1 * m13
                rdet = onef / detA
                off = g * CH + v * L
                bx[pl.ds(off, L)] = det1 * rdet
                by[pl.ds(off, L)] = det2 * rdet
                bz[pl.ds(off, L)] = det3 * rdet
            return carry

        lax.fori_loop(0, NCHUNK, chunk_body, 0)

        pltpu.sync_copy(bx, ox.at[pl.ds(base, NT)])
        pltpu.sync_copy(by, oy.at[pl.ds(base, NT)])
        pltpu.sync_copy(bz, oz.at[pl.ds(base, NT)])

    return k


def kernel(coordinates, u, connectivity_tensor):
    N, K = connectivity_tensor.shape[0], connectivity_tensor.shape[1]
    table = jnp.concatenate([coordinates, u], axis=1)
    conn = connectivity_tensor.astype(jnp.int32).reshape(N * K * 2 // 128, 128)
    ox, oy, oz = _build(N, K)(table, conn)
    return (ox[:, None], oy[:, None], oz[:, None])


# speedup vs baseline: 12.5569x; 1.0463x over previous
import functools

import jax
import jax.numpy as jnp
from jax import lax
from jax.experimental import pallas as pl
from jax.experimental.pallas import tpu as pltpu
from jax.experimental.pallas import tpu_sc as plsc

L = 16
NC = 2
NS = 16
NW = NC * NS
CH = 64
G = 128


@functools.lru_cache(maxsize=None)
def _build(N, K):
    assert K == L, "kernel specialised to K == 16"
    NT = -(-N // NW)
    NT = -(-NT // (2 * CH)) * (2 * CH)
    NCHUNK = NT // CH
    IDX_PER_CHUNK = CH * K * 2
    NGATHER = IDX_PER_CHUNK // G
    CROWS = IDX_PER_CHUNK // 128
    ROW_W = 2 * K

    mesh = plsc.VectorSubcoreMesh(core_axis_name="c", subcore_axis_name="s")

    @functools.partial(
        pl.kernel,
        mesh=mesh,
        compiler_params=pltpu.CompilerParams(
            needs_layout_passes=False, use_tc_tiling_on_sc=False
        ),
        out_type=[jax.ShapeDtypeStruct((N,), jnp.float32)] * 3,
        scratch_types=[
            pltpu.VMEM((CROWS, 128), jnp.int32),
            pltpu.VMEM((CROWS, 128), jnp.int32),
            pltpu.VMEM((IDX_PER_CHUNK, 4), jnp.float32),
            pltpu.VMEM((IDX_PER_CHUNK, 4), jnp.float32),
            pltpu.VMEM((NT,), jnp.float32),
            pltpu.VMEM((NT,), jnp.float32),
            pltpu.VMEM((NT,), jnp.float32),
            pltpu.SemaphoreType.DMA,
            pltpu.SemaphoreType.DMA,
        ],
    )
    def k(table, conn, ox, oy, oz, cvA, cvB, rvA, rvB, bx, by, bz, semA, semB):
        wid = lax.axis_index("s") * NC + lax.axis_index("c")
        base = lax.min(wid * NT, N - NT)
        base = pl.multiple_of(base, 32)
        crow0 = (base * 2 * K) // 128

        iota = lax.iota(jnp.int32, L)
        row_base = iota * ROW_W
        fcol = [jnp.full((L,), f, jnp.int32) for f in range(4)]
        onef = jnp.full((L,), 1.0, jnp.float32)
        zerof = jnp.zeros((L,), jnp.float32)

        def load_conn(c, cv):
            cc = jnp.minimum(c, NCHUNK - 1)
            crow = pl.multiple_of(crow0 + cc * CROWS, 8)
            pltpu.sync_copy(conn.at[pl.ds(crow, CROWS)], cv)

        def fire(cv, rv, sem):
            for j in range(NGATHER):
                pltpu.async_copy(
                    table.at[cv.at[j]], rv.at[pl.ds(j * G, G)], sem
                )

        def drain(cv, rv, sem):
            for j in range(NGATHER):
                pltpu.make_async_copy(
                    table.at[cv.at[j]], rv.at[pl.ds(j * G, G)], sem
                ).wait()

        def compute(c, rv):
            for v in range(CH // L):
                a1 = zerof; b1 = zerof; c1 = zerof
                b2 = zerof; c2 = zerof; c3 = zerof
                d1 = zerof; d2 = zerof; d3 = zerof
                for kk in range(K):
                    r0 = row_base + (v * L * ROW_W + 2 * kk)
                    r1 = r0 + 1
                    x1 = plsc.load_gather(rv, [r0, fcol[0]])
                    y1 = plsc.load_gather(rv, [r0, fcol[1]])
                    z1 = plsc.load_gather(rv, [r0, fcol[2]])
                    u1 = plsc.load_gather(rv, [r0, fcol[3]])
                    x2 = plsc.load_gather(rv, [r1, fcol[0]])
                    y2 = plsc.load_gather(rv, [r1, fcol[1]])
                    z2 = plsc.load_gather(rv, [r1, fcol[2]])
                    u2 = plsc.load_gather(rv, [r1, fcol[3]])
                    dx = x1 - x2
                    dy = y1 - y2
                    dz = z1 - z2
                    du = u1 - u2
                    r2 = dx * dx + dy * dy + dz * dz
                    w2 = jnp.where(r2 == zerof, onef, onef / r2)
                    wdx = w2 * dx
                    wdy = w2 * dy
                    wdz = w2 * dz
                    wdu = w2 * du
                    a1 = a1 + wdx * dx
                    b1 = b1 + wdx * dy
                    c1 = c1 + wdx * dz
                    b2 = b2 + wdy * dy
                    c2 = c2 + wdy * dz
                    c3 = c3 + wdz * dz
                    d1 = d1 + wdu * dx
                    d2 = d2 + wdu * dy
                    d3 = d3 + wdu * dz

                m11 = b2 * c3 - c2 * c2
                m12 = b1 * c3 - c2 * c1
                m13 = b1 * c2 - b2 * c1
                detA = a1 * m11 - b1 * m12 + c1 * m13
                det1 = d1 * m11 - b1 * (d2 * c3 - c2 * d3) + c1 * (d2 * c2 - b2 * d3)
                det2 = a1 * (d2 * c3 - c2 * d3) - d1 * m12 + c1 * (b1 * d3 - d2 * c1)
                det3 = a1 * (b2 * d3 - c2 * d2) - b1 * (b1 * d3 - c1 * d2) + d1 * m13
                rdet = onef / detA
                off = c * CH + v * L
                bx[pl.ds(off, L)] = det1 * rdet
                by[pl.ds(off, L)] = det2 * rdet
                bz[pl.ds(off, L)] = det3 * rdet

        load_conn(0, cvA)
        fire(cvA, rvA, semA)
        load_conn(1, cvB)

        def pair_body(t, carry):
            c0 = 2 * t
            fire(cvB, rvB, semB)
            drain(cvA, rvA, semA)
            compute(c0, rvA)
            load_conn(c0 + 2, cvA)
            fire(cvA, rvA, semA)
            drain(cvB, rvB, semB)
            compute(c0 + 1, rvB)
            load_conn(c0 + 3, cvB)
            return carry

        lax.fori_loop(0, NCHUNK // 2, pair_body, 0)
        drain(cvA, rvA, semA)

        pltpu.sync_copy(bx, ox.at[pl.ds(base, NT)])
        pltpu.sync_copy(by, oy.at[pl.ds(base, NT)])
        pltpu.sync_copy(bz, oz.at[pl.ds(base, NT)])

    return k


def kernel(coordinates, u, connectivity_tensor):
    N, K = connectivity_tensor.shape[0], connectivity_tensor.shape[1]
    table = jnp.concatenate([coordinates, u], axis=1)
    conn = connectivity_tensor.astype(jnp.int32).reshape(N * K * 2 // 128, 128)
    ox, oy, oz = _build(N, K)(table, conn)
    return (ox[:, None], oy[:, None], oz[:, None])


# speedup vs baseline: 12.5712x; 1.0011x over previous
import functools

import jax
import jax.numpy as jnp
from jax import lax
from jax.experimental import pallas as pl
from jax.experimental.pallas import tpu as pltpu
from jax.experimental.pallas import tpu_sc as plsc

L = 16
NC = 2
NS = 16
NW = NC * NS
CH = 64
G = 128


@functools.lru_cache(maxsize=None)
def _build(N, K):
    assert K == L, "kernel specialised to K == 16"
    NT = -(-N // NW)
    NT = -(-NT // (2 * CH)) * (2 * CH)
    NCHUNK = NT // CH
    IDX_PER_CHUNK = CH * K * 2
    NGATHER = IDX_PER_CHUNK // G
    CROWS = IDX_PER_CHUNK // 128
    ROW_W = 2 * K

    mesh = plsc.VectorSubcoreMesh(core_axis_name="c", subcore_axis_name="s")

    @functools.partial(
        pl.kernel,
        mesh=mesh,
        compiler_params=pltpu.CompilerParams(
            needs_layout_passes=False, use_tc_tiling_on_sc=False
        ),
        out_type=[jax.ShapeDtypeStruct((N,), jnp.float32)] * 3,
        scratch_types=[
            pltpu.VMEM((IDX_PER_CHUNK,), jnp.int32),
            pltpu.VMEM((IDX_PER_CHUNK,), jnp.int32),
            pltpu.VMEM((IDX_PER_CHUNK, 4), jnp.float32),
            pltpu.VMEM((IDX_PER_CHUNK, 4), jnp.float32),
            pltpu.VMEM((NT,), jnp.float32),
            pltpu.VMEM((NT,), jnp.float32),
            pltpu.VMEM((NT,), jnp.float32),
            pltpu.SemaphoreType.DMA,
            pltpu.SemaphoreType.DMA,
        ],
    )
    def k(table, conn, ox, oy, oz, cvA, cvB, rvA, rvB, bx, by, bz, semA, semB):
        wid = lax.axis_index("s") * NC + lax.axis_index("c")
        base = lax.min(wid * NT, N - NT)
        base = pl.multiple_of(base, 32)
        cidx0 = base * 2 * K

        iota = lax.iota(jnp.int32, L)
        row_base = iota * ROW_W
        fcol = [jnp.full((L,), f, jnp.int32) for f in range(4)]
        onef = jnp.full((L,), 1.0, jnp.float32)
        zerof = jnp.zeros((L,), jnp.float32)

        def load_conn(c, cv):
            cc = jnp.minimum(c, NCHUNK - 1)
            coff = pl.multiple_of(cidx0 + cc * IDX_PER_CHUNK, 8)
            pltpu.sync_copy(conn.at[pl.ds(coff, IDX_PER_CHUNK)], cv)

        def fire(cv, rv, sem):
            pltpu.async_copy(table.at[cv], rv, sem)

        def drain(cv, rv, sem):
            pltpu.make_async_copy(table.at[cv], rv, sem).wait()

        def compute(c, rv):
            for v in range(CH // L):
                a1 = zerof; b1 = zerof; c1 = zerof
                b2 = zerof; c2 = zerof; c3 = zerof
                d1 = zerof; d2 = zerof; d3 = zerof
                for kk in range(K):
                    r0 = row_base + (v * L * ROW_W + 2 * kk)
                    r1 = r0 + 1
                    x1 = plsc.load_gather(rv, [r0, fcol[0]])
                    y1 = plsc.load_gather(rv, [r0, fcol[1]])
                    z1 = plsc.load_gather(rv, [r0, fcol[2]])
                    u1 = plsc.load_gather(rv, [r0, fcol[3]])
                    x2 = plsc.load_gather(rv, [r1, fcol[0]])
                    y2 = plsc.load_gather(rv, [r1, fcol[1]])
                    z2 = plsc.load_gather(rv, [r1, fcol[2]])
                    u2 = plsc.load_gather(rv, [r1, fcol[3]])
                    dx = x1 - x2
                    dy = y1 - y2
                    dz = z1 - z2
                    du = u1 - u2
                    r2 = dx * dx + dy * dy + dz * dz
                    w2 = jnp.where(r2 == zerof, onef, onef / r2)
                    wdx = w2 * dx
                    wdy = w2 * dy
                    wdz = w2 * dz
                    wdu = w2 * du
                    a1 = a1 + wdx * dx
                    b1 = b1 + wdx * dy
                    c1 = c1 + wdx * dz
                    b2 = b2 + wdy * dy
                    c2 = c2 + wdy * dz
                    c3 = c3 + wdz * dz
                    d1 = d1 + wdu * dx
                    d2 = d2 + wdu * dy
                    d3 = d3 + wdu * dz

                m11 = b2 * c3 - c2 * c2
                m12 = b1 * c3 - c2 * c1
                m13 = b1 * c2 - b2 * c1
                detA = a1 * m11 - b1 * m12 + c1 * m13
                det1 = d1 * m11 - b1 * (d2 * c3 - c2 * d3) + c1 * (d2 * c2 - b2 * d3)
                det2 = a1 * (d2 * c3 - c2 * d3) - d1 * m12 + c1 * (b1 * d3 - d2 * c1)
                det3 = a1 * (b2 * d3 - c2 * d2) - b1 * (b1 * d3 - c1 * d2) + d1 * m13
                rdet = onef / detA
                off = c * CH + v * L
                bx[pl.ds(off, L)] = det1 * rdet
                by[pl.ds(off, L)] = det2 * rdet
                bz[pl.ds(off, L)] = det3 * rdet

        load_conn(0, cvA)
        fire(cvA, rvA, semA)
        load_conn(1, cvB)

        def pair_body(t, carry):
            c0 = 2 * t
            fire(cvB, rvB, semB)
            drain(cvA, rvA, semA)
            compute(c0, rvA)
            load_conn(c0 + 2, cvA)
            fire(cvA, rvA, semA)
            drain(cvB, rvB, semB)
            compute(c0 + 1, rvB)
            load_conn(c0 + 3, cvB)
            return carry

        lax.fori_loop(0, NCHUNK // 2, pair_body, 0)
        drain(cvA, rvA, semA)

        pltpu.sync_copy(bx, ox.at[pl.ds(base, NT)])
        pltpu.sync_copy(by, oy.at[pl.ds(base, NT)])
        pltpu.sync_copy(bz, oz.at[pl.ds(base, NT)])

    return k


def kernel(coordinates, u, connectivity_tensor):
    N, K = connectivity_tensor.shape[0], connectivity_tensor.shape[1]
    table = jnp.concatenate([coordinates, u], axis=1)
    conn = connectivity_tensor.astype(jnp.int32).reshape(N * K * 2)
    ox, oy, oz = _build(N, K)(table, conn)
    return (ox[:, None], oy[:, None], oz[:, None])


# speedup vs baseline: 12.5920x; 1.0017x over previous
import functools

import jax
import jax.numpy as jnp
from jax import lax
from jax.experimental import pallas as pl
from jax.experimental.pallas import tpu as pltpu
from jax.experimental.pallas import tpu_sc as plsc

L = 16
NC = 2
NS = 16
NW = NC * NS
CH = 64
G = 128


@functools.lru_cache(maxsize=None)
def _build(N, K):
    assert K == L, "kernel specialised to K == 16"
    NT = -(-N // NW)
    NT = -(-NT // (2 * CH)) * (2 * CH)
    NCHUNK = NT // CH
    IDX_PER_CHUNK = CH * K * 2
    NGATHER = IDX_PER_CHUNK // G
    CROWS = IDX_PER_CHUNK // 128
    ROW_W = 2 * K

    mesh = plsc.VectorSubcoreMesh(core_axis_name="c", subcore_axis_name="s")

    @functools.partial(
        pl.kernel,
        mesh=mesh,
        compiler_params=pltpu.CompilerParams(
            needs_layout_passes=False, use_tc_tiling_on_sc=False
        ),
        out_type=[jax.ShapeDtypeStruct((N,), jnp.float32)] * 3,
        scratch_types=[
            pltpu.VMEM((IDX_PER_CHUNK,), jnp.int32),
            pltpu.VMEM((IDX_PER_CHUNK,), jnp.int32),
            pltpu.VMEM((IDX_PER_CHUNK, 4), jnp.float32),
            pltpu.VMEM((IDX_PER_CHUNK, 4), jnp.float32),
            pltpu.VMEM((NT,), jnp.float32),
            pltpu.VMEM((NT,), jnp.float32),
            pltpu.VMEM((NT,), jnp.float32),
            pltpu.VMEM_SHARED((N, 4), jnp.float32),
            pltpu.SemaphoreType.DMA,
            pltpu.SemaphoreType.DMA,
        ],
    )
    def k(table, conn, ox, oy, oz, cvA, cvB, rvA, rvB, bx, by, bz, shr, semA, semB):
        wid = lax.axis_index("s") * NC + lax.axis_index("c")
        base = lax.min(wid * NT, N - NT)
        base = pl.multiple_of(base, 32)
        cidx0 = base * 2 * K

        iota = lax.iota(jnp.int32, L)
        row_base = iota * ROW_W
        fcol = [jnp.full((L,), f, jnp.int32) for f in range(4)]
        onef = jnp.full((L,), 1.0, jnp.float32)
        zerof = jnp.zeros((L,), jnp.float32)

        def load_conn(c, cv):
            cc = jnp.minimum(c, NCHUNK - 1)
            coff = pl.multiple_of(cidx0 + cc * IDX_PER_CHUNK, 8)
            pltpu.sync_copy(conn.at[pl.ds(coff, IDX_PER_CHUNK)], cv)

        def fire(cv, rv, sem):
            pltpu.async_copy(shr.at[cv], rv, sem)

        def drain(cv, rv, sem):
            pltpu.make_async_copy(shr.at[cv], rv, sem).wait()

        def compute(c, rv):
            for v in range(CH // L):
                a1 = zerof; b1 = zerof; c1 = zerof
                b2 = zerof; c2 = zerof; c3 = zerof
                d1 = zerof; d2 = zerof; d3 = zerof
                for kk in range(K):
                    r0 = row_base + (v * L * ROW_W + 2 * kk)
                    r1 = r0 + 1
                    x1 = plsc.load_gather(rv, [r0, fcol[0]])
                    y1 = plsc.load_gather(rv, [r0, fcol[1]])
                    z1 = plsc.load_gather(rv, [r0, fcol[2]])
                    u1 = plsc.load_gather(rv, [r0, fcol[3]])
                    x2 = plsc.load_gather(rv, [r1, fcol[0]])
                    y2 = plsc.load_gather(rv, [r1, fcol[1]])
                    z2 = plsc.load_gather(rv, [r1, fcol[2]])
                    u2 = plsc.load_gather(rv, [r1, fcol[3]])
                    dx = x1 - x2
                    dy = y1 - y2
                    dz = z1 - z2
                    du = u1 - u2
                    r2 = dx * dx + dy * dy + dz * dz
                    w2 = jnp.where(r2 == zerof, onef, onef / r2)
                    wdx = w2 * dx
                    wdy = w2 * dy
                    wdz = w2 * dz
                    wdu = w2 * du
                    a1 = a1 + wdx * dx
                    b1 = b1 + wdx * dy
                    c1 = c1 + wdx * dz
                    b2 = b2 + wdy * dy
                    c2 = c2 + wdy * dz
                    c3 = c3 + wdz * dz
                    d1 = d1 + wdu * dx
                    d2 = d2 + wdu * dy
                    d3 = d3 + wdu * dz

                m11 = b2 * c3 - c2 * c2
                m12 = b1 * c3 - c2 * c1
                m13 = b1 * c2 - b2 * c1
                detA = a1 * m11 - b1 * m12 + c1 * m13
                det1 = d1 * m11 - b1 * (d2 * c3 - c2 * d3) + c1 * (d2 * c2 - b2 * d3)
                det2 = a1 * (d2 * c3 - c2 * d3) - d1 * m12 + c1 * (b1 * d3 - d2 * c1)
                det3 = a1 * (b2 * d3 - c2 * d2) - b1 * (b1 * d3 - c1 * d2) + d1 * m13
                rdet = onef / detA
                off = c * CH + v * L
                bx[pl.ds(off, L)] = det1 * rdet
                by[pl.ds(off, L)] = det2 * rdet
                bz[pl.ds(off, L)] = det3 * rdet

        @pl.when(lax.axis_index("s") == 0)
        def _stage():
            pltpu.sync_copy(table, shr)

        plsc.subcore_barrier()

        load_conn(0, cvA)
        fire(cvA, rvA, semA)
        load_conn(1, cvB)

        def pair_body(t, carry):
            c0 = 2 * t
            fire(cvB, rvB, semB)
            drain(cvA, rvA, semA)
            compute(c0, rvA)
            load_conn(c0 + 2, cvA)
            fire(cvA, rvA, semA)
            drain(cvB, rvB, semB)
            compute(c0 + 1, rvB)
            load_conn(c0 + 3, cvB)
            return carry

        lax.fori_loop(0, NCHUNK // 2, pair_body, 0)
        drain(cvA, rvA, semA)

        pltpu.sync_copy(bx, ox.at[pl.ds(base, NT)])
        pltpu.sync_copy(by, oy.at[pl.ds(base, NT)])
        pltpu.sync_copy(bz, oz.at[pl.ds(base, NT)])

    return k


def kernel(coordinates, u, connectivity_tensor):
    N, K = connectivity_tensor.shape[0], connectivity_tensor.shape[1]
    table = jnp.concatenate([coordinates, u], axis=1)
    conn = connectivity_tensor.astype(jnp.int32).reshape(N * K * 2)
    ox, oy, oz = _build(N, K)(table, conn)
    return (ox[:, None], oy[:, None], oz[:, None])
